# baseline (speedup 1.0000x reference)
import functools

import jax
import jax.numpy as jnp
from jax import lax
from jax.experimental import pallas as pl
from jax.experimental.pallas import tpu as pltpu
from jax.experimental.pallas import tpu_sc as plsc

B, NE, H, R = 128, 64, 768, 128
N = B * NE
POS = 512
NT_PAD = 16

SC_NC, SC_NS, SC_L = 2, 16, 16
NW = SC_NC * SC_NS
ROWS_PER_W = N // NW
CHUNK = 64
NCHUNK = ROWS_PER_W // CHUNK


def _tables_body(type16_ref, post_ref, w2_ref, w3_ref, b_ref, tt_ref, pt_ref):
    tt_ref[...] = lax.dot_general(
        type16_ref[...], w2_ref[...], (((1,), (1,)), ((), ())),
        preferred_element_type=jnp.float32)
    pt = lax.dot_general(
        post_ref[...], w3_ref[...], (((1,), (1,)), ((), ())),
        preferred_element_type=jnp.float32) + b_ref[...]
    pt_ref[...] = pt.astype(jnp.bfloat16)


def _tables(type16, pos_table, W_proj, b2d):
    return pl.pallas_call(
        _tables_body,
        grid=(1,),
        in_specs=[
            pl.BlockSpec((NT_PAD, H), lambda i: (0, 0)),
            pl.BlockSpec((POS, H), lambda i: (0, 0)),
            pl.BlockSpec((H, H), lambda i: (0, 1)),
            pl.BlockSpec((H, H), lambda i: (0, 2)),
            pl.BlockSpec((1, H), lambda i: (0, 0)),
        ],
        out_specs=(
            pl.BlockSpec((NT_PAD, H), lambda i: (0, 0)),
            pl.BlockSpec((POS, H), lambda i: (0, 0)),
        ),
        out_shape=(
            jax.ShapeDtypeStruct((NT_PAD, H), jnp.float32),
            jax.ShapeDtypeStruct((POS, H), jnp.bfloat16),
        ),
    )(type16, pos_table, W_proj, W_proj, b2d)


def _gather_body(pt_hbm, pidx_hbm, out_hbm, idx_v, rows_v, sem):
    wid = lax.axis_index("s") * SC_NC + lax.axis_index("c")
    for c in range(NCHUNK):
        base = wid * ROWS_PER_W + c * CHUNK
        pltpu.sync_copy(pidx_hbm.at[pl.ds(base, CHUNK)], idx_v)
        pltpu.async_copy(pt_hbm.at[idx_v], rows_v, sem).wait()
        pltpu.sync_copy(rows_v, out_hbm.at[pl.ds(base, CHUNK)])


def _sc_gather(pt_i32, pidx):
    k = functools.partial(
        pl.kernel,
        mesh=plsc.VectorSubcoreMesh(core_axis_name="c", subcore_axis_name="s"),
        out_type=jax.ShapeDtypeStruct((N, H // 2), jnp.int32),
        scratch_types=[
            pltpu.VMEM((CHUNK,), jnp.int32),
            pltpu.VMEM((CHUNK, H // 2), jnp.int32),
            pltpu.SemaphoreType.DMA,
        ],
    )(_gather_body)
    return k(pt_i32, pidx)


MROWS = 512


def _main_body(x_ref, g_ref, oh_ref, w1_ref, tt_ref, gamma_ref, beta_ref,
               o_ref):
    acc = lax.dot_general(
        x_ref[...], w1_ref[...], (((1,), (1,)), ((), ())),
        preferred_element_type=jnp.float32)
    acc = acc + lax.dot_general(
        oh_ref[...], tt_ref[...], (((1,), (0,)), ((), ())),
        preferred_element_type=jnp.float32)
    acc = acc + g_ref[...].astype(jnp.float32)
    m = jnp.mean(acc, axis=-1, keepdims=True)
    d = acc - m
    v = jnp.mean(d * d, axis=-1, keepdims=True)
    o_ref[...] = d * lax.rsqrt(v + 1e-5) * gamma_ref[...] + beta_ref[...]


def _main(x, g, oh, W_proj, tt, gamma2d, beta2d):
    return pl.pallas_call(
        _main_body,
        grid=(N // MROWS,),
        in_specs=[
            pl.BlockSpec((MROWS, H), lambda i: (i, 0)),
            pl.BlockSpec((MROWS, H), lambda i: (i, 0)),
            pl.BlockSpec((MROWS, NT_PAD), lambda i: (i, 0)),
            pl.BlockSpec((H, H), lambda i: (0, 0)),
            pl.BlockSpec((NT_PAD, H), lambda i: (0, 0)),
            pl.BlockSpec((1, H), lambda i: (0, 0)),
            pl.BlockSpec((1, H), lambda i: (0, 0)),
        ],
        out_specs=pl.BlockSpec((MROWS, H), lambda i: (i, 0)),
        out_shape=jax.ShapeDtypeStruct((N, H), jnp.float32),
        compiler_params=pltpu.CompilerParams(
            dimension_semantics=("parallel",)),
    )(x, g, oh, W_proj, tt, gamma2d, beta2d)


NEDGE = 2 * R + NE


def _edge_body(h_ref, t_ref, r_ref, rs_ref, rd_ref, sp_ref, et_ref, bi_ref):
    off = lax.broadcasted_iota(jnp.int32, (B, R), 0) * NE
    h = h_ref[...] + off
    t = t_ref[...] + off
    rs_ref[0] = h
    rs_ref[1] = t
    rd_ref[0] = t
    rd_ref[1] = h
    j2 = (lax.broadcasted_iota(jnp.int32, (B, NE // 2), 1) * 2
          + lax.broadcasted_iota(jnp.int32, (B, NE // 2), 0) * NE)
    sp_ref[0] = j2
    sp_ref[1] = j2 + 1
    rt = r_ref[...]
    et_ref[0] = rt
    et_ref[1] = rt
    bi_ref[...] = lax.broadcasted_iota(jnp.int32, (B, NE), 0)


def _edges(rel_head, rel_tail, rel_type):
    return pl.pallas_call(
        _edge_body,
        out_shape=(
            jax.ShapeDtypeStruct((2, B, R), jnp.int32),
            jax.ShapeDtypeStruct((2, B, R), jnp.int32),
            jax.ShapeDtypeStruct((2, B, NE // 2), jnp.int32),
            jax.ShapeDtypeStruct((2, B, R), jnp.int32),
            jax.ShapeDtypeStruct((B, NE), jnp.int32),
        ),
    )(rel_head, rel_tail, rel_type)


def kernel(entity_repr, entity_type, entity_pos, rel_head, rel_tail,
           rel_type, sequence_output, W_proj, b_proj, type_table, pos_table,
           ln_gamma, ln_beta):
    type16 = jnp.zeros((NT_PAD, H), jnp.float32).at[:9].set(type_table)
    b2d = b_proj.reshape(1, H)
    gamma2d = ln_gamma.reshape(1, H)
    beta2d = ln_beta.reshape(1, H)
    tidx = entity_type.reshape(N).astype(jnp.int32)
    pidx = entity_pos.reshape(N).astype(jnp.int32)
    oh = (tidx[:, None] == jnp.arange(NT_PAD, dtype=jnp.int32)).astype(
        jnp.float32)
    x = entity_repr.reshape(N, H)

    tt, pt = _tables(type16, pos_table, W_proj, b2d)
    pt_i32 = lax.bitcast_convert_type(
        pt.reshape(POS, H // 2, 2), jnp.int32)
    g_i32 = _sc_gather(pt_i32, pidx)
    g = lax.bitcast_convert_type(g_i32, jnp.bfloat16).reshape(N, H)
    node_features = _main(x, g, oh, W_proj, tt, gamma2d, beta2d)

    rs, rd, sp, et, bi = _edges(rel_head.astype(jnp.int32),
                                rel_tail.astype(jnp.int32),
                                rel_type.astype(jnp.int32))
    idt = rel_head.dtype
    src = jnp.concatenate(
        [rs.transpose(1, 2, 0).reshape(B, 2 * R),
         sp.transpose(1, 2, 0).reshape(B, NE)], axis=1).reshape(-1)
    dst = jnp.concatenate(
        [rd.transpose(1, 2, 0).reshape(B, 2 * R),
         sp.transpose(1, 2, 0).reshape(B, NE)], axis=1).reshape(-1)
    edge_index = jnp.stack([src, dst]).astype(idt)
    edge_type = jnp.concatenate(
        [et.transpose(1, 2, 0).reshape(B, 2 * R),
         jnp.zeros((B, NE), jnp.int32)], axis=1).reshape(-1).astype(
             rel_type.dtype)
    batch_indices = bi.reshape(-1)
    return node_features, edge_index, edge_type, batch_indices


# speedup vs baseline: 2.3319x; 2.3319x over previous
import functools

import jax
import jax.numpy as jnp
from jax import lax
from jax.experimental import pallas as pl
from jax.experimental.pallas import tpu as pltpu
from jax.experimental.pallas import tpu_sc as plsc

B, NE, H, R = 128, 64, 768, 128
N = B * NE
POS = 512
NT_PAD = 16

SC_NC, SC_NS, SC_L = 2, 16, 16
NW = SC_NC * SC_NS
ROWS_PER_W = N // NW
CHUNK = 64
NCHUNK = ROWS_PER_W // CHUNK


def _tables_body(type16_ref, post_ref, w2_ref, w3_ref, b_ref, tt_ref, pt_ref):
    tt_ref[...] = lax.dot_general(
        type16_ref[...], w2_ref[...], (((1,), (1,)), ((), ())),
        preferred_element_type=jnp.float32)
    pt = lax.dot_general(
        post_ref[...], w3_ref[...], (((1,), (1,)), ((), ())),
        preferred_element_type=jnp.float32) + b_ref[...]
    u = lax.bitcast_convert_type(pt, jnp.int32)
    r = u + 0x7FFF + jnp.bitwise_and(lax.shift_right_logical(u, 16), 1)
    lo = lax.shift_right_logical(r[:, :H // 2], 16)
    hi = jnp.bitwise_and(r[:, H // 2:], jnp.int32(-65536))
    pt_ref[...] = jnp.bitwise_or(hi, lo)


def _tables(type16, pos_table, W_proj, b2d):
    return pl.pallas_call(
        _tables_body,
        grid=(1,),
        in_specs=[
            pl.BlockSpec((NT_PAD, H), lambda i: (0, 0)),
            pl.BlockSpec((POS, H), lambda i: (0, 0)),
            pl.BlockSpec((H, H), lambda i: (0, 1)),
            pl.BlockSpec((H, H), lambda i: (0, 2)),
            pl.BlockSpec((1, H), lambda i: (0, 0)),
        ],
        out_specs=(
            pl.BlockSpec((NT_PAD, H), lambda i: (0, 0)),
            pl.BlockSpec((POS, H // 2), lambda i: (0, 0)),
        ),
        out_shape=(
            jax.ShapeDtypeStruct((NT_PAD, H), jnp.float32),
            jax.ShapeDtypeStruct((POS, H // 2), jnp.int32),
        ),
    )(type16, pos_table, W_proj, W_proj, b2d)


def _gather_body(pt_hbm, pidx_hbm, out_hbm, idx_v, rows_v, sem):
    wid = lax.axis_index("s") * SC_NC + lax.axis_index("c")
    for c in range(NCHUNK):
        base = wid * ROWS_PER_W + c * CHUNK
        pltpu.sync_copy(pidx_hbm.at[pl.ds(base, CHUNK)], idx_v)
        pltpu.async_copy(pt_hbm.at[idx_v], rows_v, sem).wait()
        pltpu.sync_copy(rows_v, out_hbm.at[pl.ds(base, CHUNK)])


def _sc_gather(pt_i32, pidx):
    k = functools.partial(
        pl.kernel,
        mesh=plsc.VectorSubcoreMesh(core_axis_name="c", subcore_axis_name="s"),
        out_type=jax.ShapeDtypeStruct((N, H // 2), jnp.int32),
        scratch_types=[
            pltpu.VMEM((CHUNK,), jnp.int32),
            pltpu.VMEM((CHUNK, H // 2), jnp.int32),
            pltpu.SemaphoreType.DMA,
        ],
    )(_gather_body)
    return k(pt_i32, pidx)


MROWS = 512


def _main_body(x_ref, g_ref, oh_ref, w1_ref, tt_ref, gamma_ref, beta_ref,
               o_ref):
    acc = lax.dot_general(
        x_ref[...], w1_ref[...], (((1,), (1,)), ((), ())),
        preferred_element_type=jnp.float32)
    acc = acc + lax.dot_general(
        oh_ref[...], tt_ref[...], (((1,), (0,)), ((), ())),
        preferred_element_type=jnp.float32)
    w = g_ref[...]
    lo_f = lax.bitcast_convert_type(lax.shift_left(w, 16), jnp.float32)
    hi_f = lax.bitcast_convert_type(
        jnp.bitwise_and(w, jnp.int32(-65536)), jnp.float32)
    acc = acc + jnp.concatenate([lo_f, hi_f], axis=1)
    m = jnp.mean(acc, axis=-1, keepdims=True)
    d = acc - m
    v = jnp.mean(d * d, axis=-1, keepdims=True)
    o_ref[...] = d * lax.rsqrt(v + 1e-5) * gamma_ref[...] + beta_ref[...]


def _main(x, g, oh, W_proj, tt, gamma2d, beta2d):
    return pl.pallas_call(
        _main_body,
        grid=(N // MROWS,),
        in_specs=[
            pl.BlockSpec((MROWS, H), lambda i: (i, 0)),
            pl.BlockSpec((MROWS, H // 2), lambda i: (i, 0)),
            pl.BlockSpec((MROWS, NT_PAD), lambda i: (i, 0)),
            pl.BlockSpec((H, H), lambda i: (0, 0)),
            pl.BlockSpec((NT_PAD, H), lambda i: (0, 0)),
            pl.BlockSpec((1, H), lambda i: (0, 0)),
            pl.BlockSpec((1, H), lambda i: (0, 0)),
        ],
        out_specs=pl.BlockSpec((MROWS, H), lambda i: (i, 0)),
        out_shape=jax.ShapeDtypeStruct((N, H), jnp.float32),
        compiler_params=pltpu.CompilerParams(
            dimension_semantics=("parallel",)),
    )(x, g, oh, W_proj, tt, gamma2d, beta2d)


NEDGE = 2 * R + NE


def _edge_body(h_ref, t_ref, r_ref, rs_ref, rd_ref, sp_ref, et_ref, bi_ref):
    off = lax.broadcasted_iota(jnp.int32, (B, R), 0) * NE
    h = h_ref[...] + off
    t = t_ref[...] + off
    rs_ref[0] = h
    rs_ref[1] = t
    rd_ref[0] = t
    rd_ref[1] = h
    j2 = (lax.broadcasted_iota(jnp.int32, (B, NE // 2), 1) * 2
          + lax.broadcasted_iota(jnp.int32, (B, NE // 2), 0) * NE)
    sp_ref[0] = j2
    sp_ref[1] = j2 + 1
    rt = r_ref[...]
    et_ref[0] = rt
    et_ref[1] = rt
    bi_ref[...] = lax.broadcasted_iota(jnp.int32, (B, NE), 0)


def _edges(rel_head, rel_tail, rel_type):
    return pl.pallas_call(
        _edge_body,
        out_shape=(
            jax.ShapeDtypeStruct((2, B, R), jnp.int32),
            jax.ShapeDtypeStruct((2, B, R), jnp.int32),
            jax.ShapeDtypeStruct((2, B, NE // 2), jnp.int32),
            jax.ShapeDtypeStruct((2, B, R), jnp.int32),
            jax.ShapeDtypeStruct((B, NE), jnp.int32),
        ),
    )(rel_head, rel_tail, rel_type)


def kernel(entity_repr, entity_type, entity_pos, rel_head, rel_tail,
           rel_type, sequence_output, W_proj, b_proj, type_table, pos_table,
           ln_gamma, ln_beta):
    type16 = jnp.zeros((NT_PAD, H), jnp.float32).at[:9].set(type_table)
    b2d = b_proj.reshape(1, H)
    gamma2d = ln_gamma.reshape(1, H)
    beta2d = ln_beta.reshape(1, H)
    tidx = entity_type.reshape(N).astype(jnp.int32)
    pidx = entity_pos.reshape(N).astype(jnp.int32)
    oh = (tidx[:, None] == jnp.arange(NT_PAD, dtype=jnp.int32)).astype(
        jnp.float32)
    x = entity_repr.reshape(N, H)

    tt, pt_i32 = _tables(type16, pos_table, W_proj, b2d)
    g_i32 = _sc_gather(pt_i32, pidx)
    node_features = _main(x, g_i32, oh, W_proj, tt, gamma2d, beta2d)

    rs, rd, sp, et, bi = _edges(rel_head.astype(jnp.int32),
                                rel_tail.astype(jnp.int32),
                                rel_type.astype(jnp.int32))
    idt = rel_head.dtype
    src = jnp.concatenate(
        [rs.transpose(1, 2, 0).reshape(B, 2 * R),
         sp.transpose(1, 2, 0).reshape(B, NE)], axis=1).reshape(-1)
    dst = jnp.concatenate(
        [rd.transpose(1, 2, 0).reshape(B, 2 * R),
         sp.transpose(1, 2, 0).reshape(B, NE)], axis=1).reshape(-1)
    edge_index = jnp.stack([src, dst]).astype(idt)
    edge_type = jnp.concatenate(
        [et.transpose(1, 2, 0).reshape(B, 2 * R),
         jnp.zeros((B, NE), jnp.int32)], axis=1).reshape(-1).astype(
             rel_type.dtype)
    batch_indices = bi.reshape(-1)
    return node_features, edge_index, edge_type, batch_indices
